# Initial kernel scaffold; baseline (speedup 1.0000x reference)
import jax
import jax.numpy as jnp
from jax.experimental import pallas as pl


def kernel(x, edge_index, fc1_W, fc2_W, bn1_gamma, bn1_beta, bn2_gamma, bn2_beta, pred_W, pred_b):
    raise NotImplementedError("write your pallas kernel here")


# speedup vs baseline: 8.4037x; 8.4037x over previous
import functools

import jax
import jax.numpy as jnp
from jax import lax
from jax.experimental import pallas as pl
from jax.experimental.pallas import tpu as pltpu
from jax.experimental.pallas import tpu_sc as plsc

N_NODES = 10000
N_EDGES = 320000
D = 128
N_LAYERS = 4
EPS_BN = 1e-5

NC, NS = 2, 16
NW = NC * NS
EPT = N_EDGES // NW
CH = 128
NFULL = EPT // CH
TAIL = EPT - NFULL * CH
RING = 2
R_CHUNK = 624
N_TRUNC = R_CHUNK * NS
R_TAIL = N_NODES - N_TRUNC


def _sc_segment_sum(h, src, dst, zinit):
    mesh = plsc.VectorSubcoreMesh(core_axis_name="c", subcore_axis_name="s")

    @functools.partial(
        pl.kernel,
        out_type=jax.ShapeDtypeStruct((NC, N_NODES, D), jnp.float32),
        mesh=mesh,
        scratch_types=[
            pltpu.VMEM_SHARED((N_NODES, D), jnp.float32),
            pltpu.VMEM((RING, CH), jnp.int32),
            pltpu.VMEM((RING, CH), jnp.int32),
            pltpu.VMEM((RING, CH, D), jnp.float32),
            pltpu.VMEM((TAIL,), jnp.int32),
            pltpu.VMEM((TAIL,), jnp.int32),
            pltpu.VMEM((TAIL, D), jnp.float32),
        ] + [pltpu.SemaphoreType.DMA] * RING,
    )
    def ksc(h_hbm, src_hbm, dst_hbm, zin_hbm, out_hbm,
            acc, sidx, didx, rows, tsrc, tdst, trows, *sems):
        c = lax.axis_index("c")
        s = lax.axis_index("s")
        r0 = s * R_CHUNK

        @pl.when(c == 0)
        def _():
            pltpu.sync_copy(h_hbm.at[pl.ds(r0, R_CHUNK)],
                            acc.at[pl.ds(r0, R_CHUNK)])

            @pl.when(s == 0)
            def _():
                pltpu.sync_copy(h_hbm.at[pl.ds(N_TRUNC, R_TAIL)],
                                acc.at[pl.ds(N_TRUNC, R_TAIL)])

        @pl.when(c != 0)
        def _():
            pltpu.sync_copy(zin_hbm.at[pl.ds(0, R_CHUNK)],
                            acc.at[pl.ds(r0, R_CHUNK)])

            @pl.when(s == 0)
            def _():
                pltpu.sync_copy(zin_hbm.at[pl.ds(0, R_TAIL)],
                                acc.at[pl.ds(N_TRUNC, R_TAIL)])

        plsc.subcore_barrier()

        ebase = (c * NS + s) * EPT

        def fire(g, b):
            off = ebase + g * CH
            pltpu.sync_copy(src_hbm.at[pl.ds(off, CH)], sidx.at[b])
            pltpu.sync_copy(dst_hbm.at[pl.ds(off, CH)], didx.at[b])
            pltpu.async_copy(h_hbm.at[sidx.at[b]], rows.at[b], sems[b])

        for b in range(RING):
            fire(b, b)

        @pl.loop(0, NFULL, step=RING)
        def _(g):
            for b in range(RING):
                pltpu.make_async_copy(h_hbm.at[sidx.at[b]],
                                      rows.at[b], sems[b]).wait()
                pltpu.sync_copy(rows.at[b], acc.at[didx.at[b]], add=True)
                nxt = g + RING + b

                @pl.when(nxt < NFULL)
                def _():
                    fire(nxt, b)

        toff = ebase + NFULL * CH
        pltpu.sync_copy(src_hbm.at[pl.ds(toff, TAIL)], tsrc)
        pltpu.sync_copy(dst_hbm.at[pl.ds(toff, TAIL)], tdst)
        pltpu.async_copy(h_hbm.at[tsrc], trows, sems[0]).wait()
        pltpu.sync_copy(trows, acc.at[tdst], add=True)

        plsc.subcore_barrier()
        pltpu.sync_copy(acc.at[pl.ds(r0, R_CHUNK)],
                        out_hbm.at[c, pl.ds(r0, R_CHUNK)])

        @pl.when(s == 0)
        def _():
            pltpu.sync_copy(acc.at[pl.ds(N_TRUNC, R_TAIL)],
                            out_hbm.at[c, pl.ds(N_TRUNC, R_TAIL)])

    return ksc(h, src, dst, zinit)


def _tc_layer(p0, p1, W1, W2, g1, b1, g2, b2, pW, pb, score, last):
    if last:
        outs = jax.ShapeDtypeStruct((1, D), jnp.float32)
    else:
        outs = (jax.ShapeDtypeStruct((N_NODES, D), jnp.float32),
                jax.ShapeDtypeStruct((1, D), jnp.float32))

    def body(p0_r, p1_r, W1_r, W2_r, g1_r, b1_r, g2_r, b2_r, pW_r, pb_r,
             sc_r, *o):
        m = p0_r[...] + p1_r[...]
        y = jnp.dot(m, W1_r[...], preferred_element_type=jnp.float32)
        mu = jnp.mean(y, axis=0, keepdims=True)
        yc = y - mu
        var = jnp.mean(yc * yc, axis=0, keepdims=True)
        y = jnp.maximum(g1_r[...] * yc * lax.rsqrt(var + EPS_BN) + b1_r[...],
                        0.0)
        z = jnp.dot(y, W2_r[...], preferred_element_type=jnp.float32)
        mu2 = jnp.mean(z, axis=0, keepdims=True)
        zc = z - mu2
        var2 = jnp.mean(zc * zc, axis=0, keepdims=True)
        hn = jnp.maximum(g2_r[...] * zc * lax.rsqrt(var2 + EPS_BN) + b2_r[...],
                         0.0)
        pooled = jnp.sum(hn, axis=0, keepdims=True)
        snew = (sc_r[...]
                + jnp.dot(pooled, pW_r[...], preferred_element_type=jnp.float32)
                + pb_r[...])
        if last:
            o[0][...] = snew
        else:
            o[0][...] = hn
            o[1][...] = snew

    return pl.pallas_call(body, out_shape=outs)(
        p0, p1, W1, W2, g1, b1, g2, b2, pW, pb, score)


def kernel(x, edge_index, fc1_W, fc2_W, bn1_gamma, bn1_beta,
           bn2_gamma, bn2_beta, pred_W, pred_b):
    src = edge_index[0].astype(jnp.int32)
    dst = edge_index[1].astype(jnp.int32)
    zinit = jnp.zeros((R_CHUNK, D), jnp.float32)
    score = jnp.zeros((1, D), jnp.float32)
    h = x
    for l in range(N_LAYERS):
        parts = _sc_segment_sum(h, src, dst, zinit)
        args = (parts[0], parts[1], fc1_W[l], fc2_W[l],
                bn1_gamma[l].reshape(1, D), bn1_beta[l].reshape(1, D),
                bn2_gamma[l].reshape(1, D), bn2_beta[l].reshape(1, D),
                pred_W[l], pred_b[l].reshape(1, D), score)
        if l < N_LAYERS - 1:
            h, score = _tc_layer(*args, last=False)
        else:
            score = _tc_layer(*args, last=True)
    return score
